# baseline (device time: 41881 ns/iter reference)
import jax
import jax.numpy as jnp
from jax import lax
from jax.experimental import pallas as pl
from jax.experimental.pallas import tpu as pltpu

N_DEV = 4
B_LOC = 2
SQ = 128
SKV = 128
HG = 4
DH = 64
D_MODEL = 512
HD = HG * DH

_BF16 = jnp.bfloat16
_F32 = jnp.float32


def kernel(x, Wq, K_ext, V_ext, Wo):
    def body(x_ref, wq_ref, k_hbm, v_hbm, wo_ref, out_ref,
             wq_comm, wo_comm, k_buf, v_buf,
             kv_sems, wq_send, wq_recv, wo_send, wo_recv):
        my = lax.axis_index("i")
        right = (my + 1) % N_DEV
        left = (my - 1) % N_DEV
        b0 = my * B_LOC

        kv_copies = []
        for s in range(N_DEV):
            hg = (my - s) % N_DEV
            for j in range(HG):
                hidx = hg * HG + j
                ck = pltpu.make_async_copy(
                    k_hbm.at[pl.ds(b0, B_LOC), :, hidx, :],
                    k_buf.at[s, :, j],
                    kv_sems.at[0, s, j])
                cv = pltpu.make_async_copy(
                    v_hbm.at[pl.ds(b0, B_LOC), :, hidx, :],
                    v_buf.at[s, :, j],
                    kv_sems.at[1, s, j])
                ck.start()
                cv.start()
                kv_copies.append((ck, cv))

        wq_comm[0, :, :] = wq_ref[...].astype(_BF16)
        wo_comm[0, :, :] = wo_ref[...].astype(_BF16)

        barrier = pltpu.get_barrier_semaphore()
        for nbr in (left, right):
            pl.semaphore_signal(barrier, inc=1, device_id=(nbr,),
                                device_id_type=pl.DeviceIdType.MESH)
        pl.semaphore_wait(barrier, 2)

        qb = lax.broadcasted_iota(jnp.int32, (SQ, SKV), 0) // 64
        kb = lax.broadcasted_iota(jnp.int32, (SQ, SKV), 1) // 64
        mask_add = jnp.where(qb == kb, 0.0, -1e9).astype(_F32)

        for h in range(N_DEV):
            if h < N_DEV - 1:
                rq = pltpu.make_async_remote_copy(
                    wq_comm.at[h], wq_comm.at[h + 1],
                    wq_send.at[h], wq_recv.at[h],
                    device_id=(right,), device_id_type=pl.DeviceIdType.MESH)
                ro = pltpu.make_async_remote_copy(
                    wo_comm.at[h], wo_comm.at[h + 1],
                    wo_send.at[h], wo_recv.at[h],
                    device_id=(right,), device_id_type=pl.DeviceIdType.MESH)
                rq.start()
                ro.start()

            for j in range(HG):
                kv_copies[h * HG + j][0].wait()
                kv_copies[h * HG + j][1].wait()

            wq_s = wq_comm[h, :, :]
            wo_s = wo_comm[h, :, :]
            for b in range(B_LOC):
                xb = x_ref[b, :, :].astype(_BF16)
                q = lax.dot_general(
                    xb, wq_s, (((1,), (0,)), ((), ())),
                    preferred_element_type=_F32)
                q = (q * 0.125).astype(_BF16)
                ctx_js = []
                for j in range(HG):
                    qj = q[:, j * DH:(j + 1) * DH]
                    kj = k_buf[h, b, j, :, :].astype(_BF16)
                    sc = lax.dot_general(
                        qj, kj, (((1,), (1,)), ((), ())),
                        preferred_element_type=_F32)
                    sc = sc + mask_add
                    m = jnp.max(sc, axis=1, keepdims=True)
                    w = jnp.exp(sc - m)
                    w = w / jnp.sum(w, axis=1, keepdims=True)
                    vj = v_buf[h, b, j, :, :].astype(_BF16)
                    ctx = lax.dot_general(
                        w.astype(_BF16), vj, (((1,), (0,)), ((), ())),
                        preferred_element_type=_F32)
                    ctx_js.append(ctx.astype(_BF16))
                ctx_b = jnp.concatenate(ctx_js, axis=1)
                contrib = lax.dot_general(
                    ctx_b, wo_s, (((1,), (0,)), ((), ())),
                    preferred_element_type=_F32)
                if h == 0:
                    out_ref[b, :, :] = contrib
                else:
                    out_ref[b, :, :] = out_ref[b, :, :] + contrib

            if h < N_DEV - 1:
                rq.wait()
                ro.wait()

    return pl.pallas_call(
        body,
        out_shape=jax.ShapeDtypeStruct((B_LOC, SQ, D_MODEL), _F32),
        in_specs=[
            pl.BlockSpec(memory_space=pltpu.MemorySpace.VMEM),
            pl.BlockSpec(memory_space=pltpu.MemorySpace.VMEM),
            pl.BlockSpec(memory_space=pltpu.MemorySpace.HBM),
            pl.BlockSpec(memory_space=pltpu.MemorySpace.HBM),
            pl.BlockSpec(memory_space=pltpu.MemorySpace.VMEM),
        ],
        out_specs=pl.BlockSpec(memory_space=pltpu.MemorySpace.VMEM),
        scratch_shapes=[
            pltpu.VMEM((N_DEV, D_MODEL, HD), _BF16),
            pltpu.VMEM((N_DEV, HD, D_MODEL), _BF16),
            pltpu.VMEM((N_DEV, B_LOC, HG, SKV, DH), _F32),
            pltpu.VMEM((N_DEV, B_LOC, HG, SKV, DH), _F32),
            pltpu.SemaphoreType.DMA((2, N_DEV, HG)),
            pltpu.SemaphoreType.DMA((N_DEV - 1,)),
            pltpu.SemaphoreType.DMA((N_DEV - 1,)),
            pltpu.SemaphoreType.DMA((N_DEV - 1,)),
            pltpu.SemaphoreType.DMA((N_DEV - 1,)),
        ],
        compiler_params=pltpu.CompilerParams(collective_id=0),
    )(x, Wq, K_ext, V_ext, Wo)


# device time: 14346 ns/iter; 2.9194x vs baseline; 2.9194x over previous
import os

import jax
import jax.numpy as jnp
from jax import lax
from jax.experimental import pallas as pl
from jax.experimental.pallas import tpu as pltpu

_VARIANT = os.environ.get("SMOKE_VARIANT", "full")

N_DEV = 4
B_LOC = 2
SQ = 128
SKV = 128
HG = 4
DH = 64
D_MODEL = 512
HD = HG * DH
R = B_LOC * SQ

_BF16 = jnp.bfloat16
_F32 = jnp.float32

_REMOTE_ORDER = (1, 3, 2)


def kernel(x, Wq, K_ext, V_ext, Wo):
    def body(x_ref, wq_ref, k_hbm, v_hbm, wo_ref, out_ref,
             wq_comm, wo_comm, k_buf, v_buf, ctx_buf,
             kv_sems, wq_send, wq_recv, wo_send, wo_recv):
        my = lax.axis_index("i")
        b0 = my * B_LOC

        _do_kv = _VARIANT not in ("ringonly", "none")
        _do_comm = _VARIANT not in ("nocomm", "kvonly", "none")
        _do_compute = _VARIANT in ("full", "nocomm")

        kv_copies = {}
        if _do_kv:
            for s in range(N_DEV):
                hg = (my - s) % N_DEV
                ck = pltpu.make_async_copy(
                    k_hbm.at[pl.ds(b0, B_LOC), :, pl.ds(hg * HG, HG), :],
                    k_buf.at[s], kv_sems.at[0, s])
                cv = pltpu.make_async_copy(
                    v_hbm.at[pl.ds(b0, B_LOC), :, pl.ds(hg * HG, HG), :],
                    v_buf.at[s], kv_sems.at[1, s])
                ck.start()
                cv.start()
                kv_copies[s] = (ck, cv)

        wq_comm[0, :, :] = wq_ref[...].astype(_BF16)
        wo_comm[0, :, :] = wo_ref[...].astype(_BF16)

        barrier = pltpu.get_barrier_semaphore()
        for r in range(1, N_DEV):
            pl.semaphore_signal(barrier, inc=1, device_id=((my + r) % N_DEV,),
                                device_id_type=pl.DeviceIdType.MESH)
        pl.semaphore_wait(barrier, N_DEV - 1)

        wq_rd, wo_rd = {}, {}
        if _do_comm:
            for r in range(1, N_DEV):
                peer = (my + r) % N_DEV
                wq_rd[r] = pltpu.make_async_remote_copy(
                    wq_comm.at[0], wq_comm.at[r],
                    wq_send.at[r - 1], wq_recv.at[r - 1],
                    device_id=(peer,), device_id_type=pl.DeviceIdType.MESH)
                wo_rd[r] = pltpu.make_async_remote_copy(
                    wo_comm.at[0], wo_comm.at[r],
                    wo_send.at[r - 1], wo_recv.at[r - 1],
                    device_id=(peer,), device_id_type=pl.DeviceIdType.MESH)
                wq_rd[r].start()

        qb = lax.broadcasted_iota(jnp.int32, (R, R), 0) // 64
        kb = lax.broadcasted_iota(jnp.int32, (R, R), 1) // 64
        mask_add = jnp.where(qb == kb, 0.0, -1e9).astype(_F32)

        def attn(slot):
            wq_s = wq_comm[slot, :, :]
            xb = x_ref[...].astype(_BF16)
            q = lax.dot_general(
                xb, wq_s, (((1,), (0,)), ((), ())),
                preferred_element_type=_F32)
            q = (q * 0.125).astype(_BF16)
            ctx_js = []
            for j in range(HG):
                qj = q[:, j * DH:(j + 1) * DH]
                kj = jnp.concatenate(
                    [k_buf[slot, b, :, j, :] for b in range(B_LOC)],
                    axis=0).astype(_BF16)
                sc = lax.dot_general(
                    qj, kj, (((1,), (1,)), ((), ())),
                    preferred_element_type=_F32)
                sc = sc + mask_add
                m = jnp.max(sc, axis=1, keepdims=True)
                w = jnp.exp(sc - m)
                w = w / jnp.sum(w, axis=1, keepdims=True)
                vj = jnp.concatenate(
                    [v_buf[slot, b, :, j, :] for b in range(B_LOC)],
                    axis=0).astype(_BF16)
                ctx = lax.dot_general(
                    w.astype(_BF16), vj, (((1,), (0,)), ((), ())),
                    preferred_element_type=_F32)
                ctx_js.append(ctx.astype(_BF16))
            ctx_buf[slot, :, :] = jnp.concatenate(ctx_js, axis=1)

        def wo_dot(slot, first):
            contrib = lax.dot_general(
                ctx_buf[slot, :, :], wo_comm[slot, :, :],
                (((1,), (0,)), ((), ())),
                preferred_element_type=_F32)
            if first:
                out_ref[...] = contrib
            else:
                out_ref[...] = out_ref[...] + contrib

        if not _do_compute:
            out_ref[...] = jnp.zeros_like(out_ref)

        if _do_compute:
            if _do_kv:
                kv_copies[0][0].wait()
                kv_copies[0][1].wait()
            attn(0)
            wo_dot(0, first=True)

        if _do_comm:
            for r in range(1, N_DEV):
                wq_rd[r].wait_send()
            for r in range(1, N_DEV):
                wo_rd[r].start()

        for slot in _REMOTE_ORDER:
            if _do_kv and _do_compute:
                kv_copies[slot][0].wait()
                kv_copies[slot][1].wait()
            if _do_comm:
                wq_rd[slot].wait_recv()
            if _do_compute:
                attn(slot)

        for slot in _REMOTE_ORDER:
            if _do_comm:
                wo_rd[slot].wait_recv()
            if _do_compute:
                wo_dot(slot, first=False)

        if _do_comm:
            for r in range(1, N_DEV):
                wo_rd[r].wait_send()
        if _do_kv and not _do_compute:
            for s in range(N_DEV):
                kv_copies[s][0].wait()
                kv_copies[s][1].wait()

    out = pl.pallas_call(
        body,
        out_shape=jax.ShapeDtypeStruct((R, D_MODEL), _F32),
        in_specs=[
            pl.BlockSpec(memory_space=pltpu.MemorySpace.VMEM),
            pl.BlockSpec(memory_space=pltpu.MemorySpace.VMEM),
            pl.BlockSpec(memory_space=pltpu.MemorySpace.HBM),
            pl.BlockSpec(memory_space=pltpu.MemorySpace.HBM),
            pl.BlockSpec(memory_space=pltpu.MemorySpace.VMEM),
        ],
        out_specs=pl.BlockSpec(memory_space=pltpu.MemorySpace.VMEM),
        scratch_shapes=[
            pltpu.VMEM((N_DEV, D_MODEL, HD), _BF16),
            pltpu.VMEM((N_DEV, HD, D_MODEL), _BF16),
            pltpu.VMEM((N_DEV, B_LOC, SKV, HG, DH), _F32),
            pltpu.VMEM((N_DEV, B_LOC, SKV, HG, DH), _F32),
            pltpu.VMEM((N_DEV, R, HD), _BF16),
            pltpu.SemaphoreType.DMA((2, N_DEV)),
            pltpu.SemaphoreType.DMA((N_DEV - 1,)),
            pltpu.SemaphoreType.DMA((N_DEV - 1,)),
            pltpu.SemaphoreType.DMA((N_DEV - 1,)),
            pltpu.SemaphoreType.DMA((N_DEV - 1,)),
        ],
        compiler_params=pltpu.CompilerParams(collective_id=0),
    )(x.reshape(R, D_MODEL), Wq, K_ext, V_ext, Wo)
    return out.reshape(B_LOC, SQ, D_MODEL)
